# Initial kernel scaffold; baseline (speedup 1.0000x reference)
import jax
import jax.numpy as jnp
from jax.experimental import pallas as pl


def kernel(token_ids, weight):
    raise NotImplementedError("write your pallas kernel here")


# speedup vs baseline: 1.8759x; 1.8759x over previous
import functools

import jax
import jax.numpy as jnp
from jax import lax
from jax.experimental import pallas as pl
from jax.experimental.pallas import tpu as pltpu
from jax.experimental.pallas import tpu_sc as plsc

EMB_D = 64
GCHUNK = 128


@functools.cache
def _build_kernel(B: int, n_super: int, k_per_super: int, nw: int):
    sup = k_per_super * GCHUNK
    b_per_w = n_super * sup
    mesh = plsc.VectorSubcoreMesh(core_axis_name="c", subcore_axis_name="s")

    @functools.partial(
        pl.kernel,
        mesh=mesh,
        compiler_params=pltpu.CompilerParams(use_tc_tiling_on_sc=False),
        out_type=jax.ShapeDtypeStruct((B, EMB_D), jnp.float32),
        scratch_types=[
            pltpu.VMEM((n_super * k_per_super, GCHUNK), jnp.int32),
            pltpu.VMEM((2, sup, EMB_D), jnp.float32),
            pltpu.SemaphoreType.DMA,
            pltpu.SemaphoreType.DMA,
        ],
    )
    def emb(idx_hbm, table_hbm, out_hbm, idx_v, rows_v, gsem0, gsem1):
        cid = lax.axis_index("c")
        sid = lax.axis_index("s")
        wid = sid * 2 + cid
        base = wid * b_per_w

        pltpu.sync_copy(idx_hbm.at[wid], idx_v)

        gsems = (gsem0, gsem1)

        def start_super(g, b):
            for j in range(k_per_super):
                pltpu.async_copy(
                    table_hbm.at[idx_v.at[g * k_per_super + j]],
                    rows_v.at[b].at[pl.ds(j * GCHUNK, GCHUNK)],
                    gsems[b],
                )

        def wait_super(b):
            pltpu.make_async_copy(
                table_hbm.at[pl.ds(0, sup)], rows_v.at[b], gsems[b]
            ).wait()

        start_super(0, 0)

        def body(o, carry):
            for b in range(2):
                g = o * 2 + b
                nxt = g + 1

                @pl.when(nxt < n_super)
                def _():
                    start_super(nxt, (b + 1) % 2)

                wait_super(b)
                pltpu.sync_copy(
                    rows_v.at[b], out_hbm.at[pl.ds(base + g * sup, sup)]
                )
            return carry

        lax.fori_loop(0, n_super // 2, body, None)

    return emb


def kernel(token_ids, weight):
    orig_shape = token_ids.shape
    B = token_ids.size
    nw = 32
    k_per_super = 5
    sup = k_per_super * GCHUNK
    b_per_w = B // nw
    n_super = b_per_w // sup
    assert n_super * sup * nw == B

    idx = token_ids.reshape(nw, n_super * k_per_super, GCHUNK).astype(jnp.int32)
    out = _build_kernel(B, n_super, k_per_super, nw)(idx, weight)
    return out.reshape(*orig_shape, EMB_D)
